# baseline (speedup 1.0000x reference)
import jax
import jax.numpy as jnp
from jax.experimental import pallas as pl
from jax.experimental.pallas import tpu as pltpu

_DIM = 128
_NH = 8
_PACK = 16
_CHUNK = 4000
_NBUF = 8


def _make_body(nbuf):
    def _lsh_body(x_hbm, rvb_ref, o_ref, buf, sem):
        i = pl.program_id(0)
        nsteps = pl.num_programs(0)
        slot = jax.lax.rem(i, nbuf)

        @pl.when(i == 0)
        def _prefetch():
            for k in range(nbuf):
                pltpu.make_async_copy(
                    x_hbm.at[pl.ds(k * _CHUNK, _CHUNK), :],
                    buf.at[k],
                    sem.at[k],
                ).start()

        pltpu.make_async_copy(
            x_hbm.at[pl.ds(i * _CHUNK, _CHUNK), :],
            buf.at[slot],
            sem.at[slot],
        ).wait()

        xp = buf[slot].reshape(_CHUNK // _PACK, _PACK * _DIM)
        packed = jnp.dot(xp, rvb_ref[...], preferred_element_type=jnp.float32)
        buckets = jnp.floor(packed).astype(jnp.int32) & 1023
        o_ref[...] = buckets.astype(jnp.float32)[None]

        @pl.when(i + nbuf < nsteps)
        def _next():
            pltpu.make_async_copy(
                x_hbm.at[pl.ds((i + nbuf) * _CHUNK, _CHUNK), :],
                buf.at[slot],
                sem.at[slot],
            ).start()

    return _lsh_body


def kernel(x, random_vectors):
    n = x.shape[0]
    nsteps = n // _CHUNK
    nbuf = min(_NBUF, nsteps)
    eye = jnp.eye(_PACK, dtype=jnp.float32)
    rv_big = jnp.einsum("jk,dh->jdkh", eye, random_vectors).reshape(
        _PACK * _DIM, _PACK * _NH
    )
    out_t = pl.pallas_call(
        _make_body(nbuf),
        grid=(nsteps,),
        in_specs=[
            pl.BlockSpec(memory_space=pltpu.MemorySpace.HBM),
            pl.BlockSpec((_PACK * _DIM, _PACK * _NH), lambda i: (0, 0)),
        ],
        out_specs=pl.BlockSpec((1, _CHUNK // _PACK, 128), lambda i: (i, 0, 0)),
        out_shape=jax.ShapeDtypeStruct((nsteps, _CHUNK // _PACK, 128), jnp.float32),
        scratch_shapes=[
            pltpu.VMEM((nbuf, _CHUNK, _DIM), jnp.float32),
            pltpu.SemaphoreType.DMA((nbuf,)),
        ],
        compiler_params=pltpu.CompilerParams(
            dimension_semantics=("arbitrary",),
        ),
    )(x, random_vectors)
    return out_t.reshape(n, _NH)
